# baseline (speedup 1.0000x reference)
import functools

import jax
import jax.numpy as jnp
from jax import lax
from jax.experimental import pallas as pl
from jax.experimental.pallas import tpu as pltpu
from jax.experimental.pallas import tpu_sc as plsc

N = 100000
E = 50000
D = 128
LN_EPS = 1e-3

NC = 2
NS = 16
L = 16

NCHUNKS = 8
NSTAGES = 2
CPS = NCHUNKS // NSTAGES
CPC = CPS // NC
CB = 6400
CHUNK_LO = tuple(c * CB for c in range(NCHUNKS))
ACC_ROWS = 6408
DUMP = CB
CSW = CB // NS
LAST_REM = E - (NCHUNKS - 1) * CB
LAST_CSW = 328
LAST_TAIL = LAST_REM - (NS - 1) * LAST_CSW
STAGE_BASE = (0, CPS * CB)
STAGE_ROWS = (CPS * CB, E - CPS * CB)

SCAN_MAIN = 99840
SCAN_PER_TILE = SCAN_MAIN // NS
SCAN_TAIL = N - SCAN_MAIN
SB = 128
NBUF = 3


@functools.lru_cache(maxsize=NSTAGES)
def _make_sc_segment_sum(stage):
  mesh = plsc.VectorSubcoreMesh(core_axis_name="c", subcore_axis_name="s",
                                num_cores=NC, num_subcores=NS)
  stage_chunks = tuple(range(stage * CPS, (stage + 1) * CPS))
  edges_needed = ([CHUNK_LO[c] for c in stage_chunks]
                  + [CHUNK_LO[stage_chunks[-1]] + CB])
  thresholds = tuple(v for v in edges_needed if 0 < v < E)

  def body(nodes_hbm, ids_hbm, out_hbm,
           rows_v0, rows_v1, rows_v2, idsv0, idsv1, idsv2, idx_r,
           idscan_v, cnt_v, call_v, zeros_v,
           sem_r0, sem_r1, sem_r2, sem_i0, sem_i1, sem_i2,
           cnt_sh, acc):
    rows_bufs = (rows_v0, rows_v1, rows_v2)
    ids_bufs = (idsv0, idsv1, idsv2)
    sems_r = (sem_r0, sem_r1, sem_r2)
    sems_i = (sem_i0, sem_i1, sem_i2)
    cid = lax.axis_index("c")
    sid = lax.axis_index("s")

    zvec = jnp.zeros((L,), jnp.float32)

    def _zrow(r, carry):
      for j in range(D // L):
        zeros_v[r, pl.ds(j * L, L)] = zvec
      return carry

    lax.fori_loop(0, zeros_v.shape[0], _zrow, 0)

    base = pl.multiple_of(sid * SCAN_PER_TILE, 8)
    pltpu.sync_copy(ids_hbm.at[pl.ds(base, SCAN_PER_TILE)], idscan_v)

    one = jnp.ones((L,), jnp.int32)
    zero = jnp.zeros((L,), jnp.int32)
    nb = len(thresholds)

    def _count(i, accs):
      v = idscan_v[pl.ds(i * L, L)]
      return tuple(accs[k] + jnp.where(v < thresholds[k], one, zero)
                   for k in range(nb))

    z = jnp.zeros((L,), jnp.int32)
    cnts = lax.fori_loop(0, SCAN_PER_TILE // L, _count,
                         tuple(z for _ in range(nb)))
    for k in range(nb):
      cnt_v[pl.ds(k * L, L)] = cnts[k]
    pltpu.sync_copy(cnt_v, cnt_sh.at[sid])

    pltpu.sync_copy(ids_hbm.at[pl.ds(SCAN_MAIN, SCAN_TAIL)],
                    idscan_v.at[pl.ds(0, SCAN_TAIL)])

    def _count_tail(i, accs):
      v = idscan_v[pl.ds(i * L, L)]
      return tuple(accs[k] + jnp.where(v < thresholds[k], one, zero)
                   for k in range(nb))

    tails = lax.fori_loop(0, SCAN_TAIL // L, _count_tail,
                          tuple(z for _ in range(nb)))
    plsc.subcore_barrier()
    pltpu.sync_copy(cnt_sh, call_v)

    sums = list(tails)
    for s in range(NS):
      for k in range(nb):
        sums[k] = sums[k] + call_v[s, pl.ds(k * L, L)]
    rs = [jnp.sum(sums[k]) for k in range(nb)]
    bounds = []
    for v in edges_needed:
      if v <= 0:
        bounds.append(jnp.int32(0))
      elif v >= E:
        bounds.append(jnp.int32(N))
      else:
        bounds.append(rs[thresholds.index(v)])

    iota = lax.iota(jnp.int32, L)
    dump_vec = jnp.full((L,), DUMP, jnp.int32)

    def _wblocks(total):
      return (SB,) * (total // SB) + (
          (total % SB,) if total % SB else ())

    def _strip_sizes(c):
      if CHUNK_LO[c] + CB <= E:
        return CSW, _wblocks(CSW), _wblocks(CSW)
      return LAST_CSW, _wblocks(LAST_CSW), _wblocks(LAST_TAIL)

    def do_chunk(c):
      cc = c - stage * CPS
      v_lo = CHUNK_LO[c]
      v_out = v_lo - STAGE_BASE[stage]
      cs = CB
      lo, hi = bounds[cc], bounds[cc + 1]
      csw, sizes_main, sizes_last = _strip_sizes(c)
      woff = pl.multiple_of(sid * csw, 8)

      def _for_my_sizes(fn):
        @pl.when(sid < NS - 1)
        def _():
          fn(sizes_main)

        @pl.when(sid == NS - 1)
        def _():
          fn(sizes_last)

      def _zero(sizes):
        done = 0
        for n in sizes:
          zdone = 0
          while zdone < n:
            zn = min(n - zdone, zeros_v.shape[0])
            pltpu.sync_copy(zeros_v.at[pl.ds(0, zn)],
                            acc.at[pl.ds(woff + done + zdone, zn)])
            zdone += zn
          done += n

      _for_my_sizes(_zero)
      plsc.subcore_barrier()

      lo8 = lo - lax.rem(lo, 8)
      span = hi - lo8
      sub = ((span + 127) // 128) * 8
      a_t = lo8 + sid * sub
      b_t = a_t + sub
      nblkr = (sub + NBUF * SB - 1) // (NBUF * SB)

      def _start_for(j):
        return pl.multiple_of(jnp.minimum(a_t + j * SB, N - SB), 8)

      def _issue(j, b):
        st = _start_for(j)
        pltpu.async_copy(ids_hbm.at[pl.ds(st, SB)], ids_bufs[b], sems_i[b])
        pltpu.async_copy(nodes_hbm.at[pl.ds(st, SB)], rows_bufs[b],
                         sems_r[b])

      def _wait(b):
        pltpu.make_async_copy(ids_hbm.at[pl.ds(0, SB)], ids_bufs[b],
                              sems_i[b]).wait()
        pltpu.make_async_copy(nodes_hbm.at[pl.ds(0, SB)], rows_bufs[b],
                              sems_r[b]).wait()

      def _process(j, b):
        nominal = a_t + j * SB
        start = _start_for(j)
        for i in range(SB // L):
          v = ids_bufs[b][pl.ds(i * L, L)]
          local = v - v_lo
          rowid = iota + (start + i * L)
          m = ((local >= 0) & (local < cs)
               & (rowid >= nominal) & (rowid < b_t))
          idx = jnp.where(m, local, dump_vec)
          idx_r[0, pl.ds(i * L, L)] = idx
        pltpu.sync_copy(rows_bufs[b], acc.at[idx_r.at[0]], add=True)

      for b in range(NBUF):
        _issue(b, b)

      def _ring(jr, carry):
        j = NBUF * jr
        for b in range(NBUF):
          _wait(b)
          _process(j + b, b)
          _issue(j + b + NBUF, b)
        return carry

      lax.fori_loop(0, nblkr, _ring, 0)
      for b in range(NBUF):
        _wait(b)
      plsc.subcore_barrier()

      def _write(sizes):
        wdone = 0
        for n in sizes:
          pltpu.sync_copy(acc.at[pl.ds(woff + wdone, n)],
                          out_hbm.at[pl.ds(v_out + woff + wdone, n)])
          wdone += n

      _for_my_sizes(_write)

    for core in range(NC):
      @pl.when(cid == core)
      def _(core=core):
        for c in stage_chunks[core * CPC:(core + 1) * CPC]:
          do_chunk(c)

  return pl.kernel(
      body,
      out_type=jax.ShapeDtypeStruct((STAGE_ROWS[stage], D), jnp.float32),
      mesh=mesh,
      compiler_params=pltpu.CompilerParams(needs_layout_passes=False),
      scratch_types=[
          pltpu.VMEM((SB, D), jnp.float32),
          pltpu.VMEM((SB, D), jnp.float32),
          pltpu.VMEM((SB, D), jnp.float32),
          pltpu.VMEM((SB,), jnp.int32),
          pltpu.VMEM((SB,), jnp.int32),
          pltpu.VMEM((SB,), jnp.int32),
          pltpu.VMEM((1, 128), jnp.int32),
          pltpu.VMEM((SCAN_PER_TILE,), jnp.int32),
          pltpu.VMEM((128,), jnp.int32),
          pltpu.VMEM((NS, 128), jnp.int32),
          pltpu.VMEM((32, D), jnp.float32),
          pltpu.SemaphoreType.DMA,
          pltpu.SemaphoreType.DMA,
          pltpu.SemaphoreType.DMA,
          pltpu.SemaphoreType.DMA,
          pltpu.SemaphoreType.DMA,
          pltpu.SemaphoreType.DMA,
          pltpu.VMEM_SHARED((NS, 128), jnp.int32),
          pltpu.VMEM_SHARED((ACC_ROWS, D), jnp.float32),
      ],
  )


BR = 3200


def _mlp_body(e_ref, a_ref, g_ref, w1_ref, b1_ref, w2_ref, b2_ref,
              gm_ref, bt_ref, o_ref):
  w1 = w1_ref[...]
  x = jnp.dot(e_ref[...], w1[0:D], preferred_element_type=jnp.float32)
  x = x + jnp.dot(a_ref[...], w1[D:2 * D],
                  preferred_element_type=jnp.float32)
  g = jnp.dot(g_ref[...], w1[2 * D:3 * D],
              preferred_element_type=jnp.float32)
  h = jnp.maximum(x + g + b1_ref[...], 0.0)
  h = jnp.maximum(
      jnp.dot(h, w2_ref[...], preferred_element_type=jnp.float32)
      + b2_ref[...], 0.0)
  m = jnp.mean(h, axis=-1, keepdims=True)
  cdev = h - m
  var = jnp.mean(cdev * cdev, axis=-1, keepdims=True)
  o_ref[...] = (cdev * lax.rsqrt(var + LN_EPS)) * gm_ref[...] + bt_ref[...]


def _mlp_body_alias(p_ref, e_ref, a_ref, g_ref, w1_ref, b1_ref, w2_ref,
                    b2_ref, gm_ref, bt_ref, o_ref):
  del p_ref
  _mlp_body(e_ref, a_ref, g_ref, w1_ref, b1_ref, w2_ref, b2_ref,
            gm_ref, bt_ref, o_ref)


def _tc_mlp_stage(stage, prev, edges, agg, globals_, W1, b1, W2, b2,
                  gamma, beta):
  nblocks = pl.cdiv(STAGE_ROWS[stage], BR)
  off = STAGE_BASE[stage] // BR
  full = lambda shape: pl.BlockSpec(shape, lambda i: (0, 0))
  row_spec = pl.BlockSpec((BR, D), lambda i: (i + off, 0))
  in_specs = [
      row_spec,
      pl.BlockSpec((BR, D), lambda i: (i, 0)),
      full((1, D)),
      full((3 * D, D)),
      full((1, D)),
      full((D, D)),
      full((1, D)),
      full((1, D)),
      full((1, D)),
  ]
  args = [edges, agg, globals_, W1, b1, W2, b2, gamma, beta]
  body = _mlp_body
  aliases = {}
  if stage > 0:
    in_specs = [pl.BlockSpec(memory_space=pl.ANY)] + in_specs
    args = [prev] + args
    body = _mlp_body_alias
    aliases = {0: 0}
  return pl.pallas_call(
      body,
      grid=(nblocks,),
      in_specs=in_specs,
      out_specs=row_spec,
      out_shape=jax.ShapeDtypeStruct((E, D), jnp.float32),
      input_output_aliases=aliases,
  )(*args)


def kernel(edges, nodes, globals_, segment_ids, num, W1, b1, W2, b2,
           gamma, beta):
  del num
  row = lambda v: v.reshape(1, D)
  wargs = (globals_, W1, row(b1), W2, row(b2), row(gamma), row(beta))
  agg0 = _make_sc_segment_sum(0)(nodes, segment_ids)
  agg1 = _make_sc_segment_sum(1)(nodes, segment_ids)
  out = _tc_mlp_stage(0, None, edges, agg0, *wargs)
  return _tc_mlp_stage(1, out, edges, agg1, *wargs)


# speedup vs baseline: 1.0177x; 1.0177x over previous
import functools

import jax
import jax.numpy as jnp
from jax import lax
from jax.experimental import pallas as pl
from jax.experimental.pallas import tpu as pltpu
from jax.experimental.pallas import tpu_sc as plsc

N = 100000
E = 50000
D = 128
LN_EPS = 1e-3

NC = 2
NS = 16
L = 16

NCHUNKS = 8
CB = 6400
CHUNK_LO = tuple(c * CB for c in range(NCHUNKS))
ACC_ROWS = 6408
DUMP = CB
CSW = CB // NS
LAST_REM = E - (NCHUNKS - 1) * CB
LAST_CSW = 328
LAST_TAIL = LAST_REM - (NS - 1) * LAST_CSW
STAGE_CHUNKS = ((0, 1, 2, 3), (4, 5), (6, 7))
NSTAGES = len(STAGE_CHUNKS)
STAGE_BASE = tuple(CHUNK_LO[sc[0]] for sc in STAGE_CHUNKS)
STAGE_ROWS = tuple(
    min(E, CHUNK_LO[sc[-1]] + CB) - CHUNK_LO[sc[0]] for sc in STAGE_CHUNKS)

SCAN_MAIN = 99840
SCAN_PER_TILE = SCAN_MAIN // NS
SCAN_TAIL = N - SCAN_MAIN
SB = 128
NBUF = 3


@functools.lru_cache(maxsize=NSTAGES)
def _make_sc_segment_sum(stage):
  mesh = plsc.VectorSubcoreMesh(core_axis_name="c", subcore_axis_name="s",
                                num_cores=NC, num_subcores=NS)
  stage_chunks = STAGE_CHUNKS[stage]
  edges_needed = ([CHUNK_LO[c] for c in stage_chunks]
                  + [CHUNK_LO[stage_chunks[-1]] + CB])
  thresholds = (tuple(k * CB for k in range(1, NCHUNKS) if k * CB < E)
                if stage == 0 else ())

  def body(*refs):
    if stage == 0:
      (nodes_hbm, ids_hbm, out_hbm, rs_hbm,
       rows_v0, rows_v1, rows_v2, idsv0, idsv1, idsv2, idx_r,
       idscan_v, cnt_v, call_v, zeros_v,
       sem_r0, sem_r1, sem_r2, sem_i0, sem_i1, sem_i2,
       cnt_sh, acc) = refs
    else:
      (nodes_hbm, ids_hbm, rs_hbm, out_hbm,
       rows_v0, rows_v1, rows_v2, idsv0, idsv1, idsv2, idx_r,
       cnt_v, call_v, zeros_v,
       sem_r0, sem_r1, sem_r2, sem_i0, sem_i1, sem_i2,
       cnt_sh, acc) = refs
    rows_bufs = (rows_v0, rows_v1, rows_v2)
    ids_bufs = (idsv0, idsv1, idsv2)
    sems_r = (sem_r0, sem_r1, sem_r2)
    sems_i = (sem_i0, sem_i1, sem_i2)
    cid = lax.axis_index("c")
    sid = lax.axis_index("s")

    zvec = jnp.zeros((L,), jnp.float32)

    def _zrow(r, carry):
      for j in range(D // L):
        zeros_v[r, pl.ds(j * L, L)] = zvec
      return carry

    lax.fori_loop(0, zeros_v.shape[0], _zrow, 0)

    one = jnp.ones((L,), jnp.int32)
    zero = jnp.zeros((L,), jnp.int32)
    z = jnp.zeros((L,), jnp.int32)

    if stage == 0:
      nb = len(thresholds)
      base = pl.multiple_of(sid * SCAN_PER_TILE, 8)
      pltpu.sync_copy(ids_hbm.at[pl.ds(base, SCAN_PER_TILE)], idscan_v)

      def _count(i, accs):
        v = idscan_v[pl.ds(i * L, L)]
        return tuple(accs[k] + jnp.where(v < thresholds[k], one, zero)
                     for k in range(nb))

      cnts = lax.fori_loop(0, SCAN_PER_TILE // L, _count,
                           tuple(z for _ in range(nb)))
      for k in range(nb):
        cnt_v[pl.ds(k * L, L)] = cnts[k]
      pltpu.sync_copy(cnt_v, cnt_sh.at[sid])

      pltpu.sync_copy(ids_hbm.at[pl.ds(SCAN_MAIN, SCAN_TAIL)],
                      idscan_v.at[pl.ds(0, SCAN_TAIL)])

      def _count_tail(i, accs):
        v = idscan_v[pl.ds(i * L, L)]
        return tuple(accs[k] + jnp.where(v < thresholds[k], one, zero)
                     for k in range(nb))

      tails = lax.fori_loop(0, SCAN_TAIL // L, _count_tail,
                            tuple(z for _ in range(nb)))
      plsc.subcore_barrier()
      pltpu.sync_copy(cnt_sh, call_v)

      sums = list(tails)
      for s in range(NS):
        for k in range(nb):
          sums[k] = sums[k] + call_v[s, pl.ds(k * L, L)]
      totals = [jnp.sum(sums[k]) for k in range(nb)]
      for k in range(nb):
        cnt_v[pl.ds(k * L, L)] = totals[k] + z

      @pl.when((cid == 0) & (sid == 0))
      def _():
        pltpu.sync_copy(cnt_v, rs_hbm)

      def _boundary(k):
        return totals[k]
    else:
      pltpu.sync_copy(rs_hbm, cnt_v)

      def _boundary(k):
        return cnt_v[pl.ds(k * L, L)][0]

    bounds = []
    for v in edges_needed:
      if v <= 0:
        bounds.append(jnp.int32(0))
      elif v >= E:
        bounds.append(jnp.int32(N))
      else:
        bounds.append(_boundary(v // CB - 1))

    iota = lax.iota(jnp.int32, L)
    dump_vec = jnp.full((L,), DUMP, jnp.int32)

    def _wblocks(total):
      return (SB,) * (total // SB) + (
          (total % SB,) if total % SB else ())

    def _strip_sizes(c):
      if CHUNK_LO[c] + CB <= E:
        return CSW, _wblocks(CSW), _wblocks(CSW)
      return LAST_CSW, _wblocks(LAST_CSW), _wblocks(LAST_TAIL)

    def do_chunk(c):
      cc = stage_chunks.index(c)
      v_lo = CHUNK_LO[c]
      v_out = v_lo - STAGE_BASE[stage]
      cs = CB
      lo, hi = bounds[cc], bounds[cc + 1]
      csw, sizes_main, sizes_last = _strip_sizes(c)
      woff = pl.multiple_of(sid * csw, 8)

      def _for_my_sizes(fn):
        @pl.when(sid < NS - 1)
        def _():
          fn(sizes_main)

        @pl.when(sid == NS - 1)
        def _():
          fn(sizes_last)

      def _zero(sizes):
        done = 0
        for n in sizes:
          zdone = 0
          while zdone < n:
            zn = min(n - zdone, zeros_v.shape[0])
            pltpu.sync_copy(zeros_v.at[pl.ds(0, zn)],
                            acc.at[pl.ds(woff + done + zdone, zn)])
            zdone += zn
          done += n

      _for_my_sizes(_zero)
      plsc.subcore_barrier()

      lo8 = lo - lax.rem(lo, 8)
      span = hi - lo8
      sub = ((span + 127) // 128) * 8
      a_t = lo8 + sid * sub
      b_t = a_t + sub
      nblkr = (sub + NBUF * SB - 1) // (NBUF * SB)

      def _start_for(j):
        return pl.multiple_of(jnp.minimum(a_t + j * SB, N - SB), 8)

      def _issue(j, b):
        st = _start_for(j)
        pltpu.async_copy(ids_hbm.at[pl.ds(st, SB)], ids_bufs[b], sems_i[b])
        pltpu.async_copy(nodes_hbm.at[pl.ds(st, SB)], rows_bufs[b],
                         sems_r[b])

      def _wait(b):
        pltpu.make_async_copy(ids_hbm.at[pl.ds(0, SB)], ids_bufs[b],
                              sems_i[b]).wait()
        pltpu.make_async_copy(nodes_hbm.at[pl.ds(0, SB)], rows_bufs[b],
                              sems_r[b]).wait()

      def _process(j, b):
        nominal = a_t + j * SB
        start = _start_for(j)
        for i in range(SB // L):
          v = ids_bufs[b][pl.ds(i * L, L)]
          local = v - v_lo
          rowid = iota + (start + i * L)
          m = ((local >= 0) & (local < cs)
               & (rowid >= nominal) & (rowid < b_t))
          idx = jnp.where(m, local, dump_vec)
          idx_r[0, pl.ds(i * L, L)] = idx
        pltpu.sync_copy(rows_bufs[b], acc.at[idx_r.at[0]], add=True)

      for b in range(NBUF):
        _issue(b, b)

      def _ring(jr, carry):
        j = NBUF * jr
        for b in range(NBUF):
          _wait(b)
          _process(j + b, b)
          _issue(j + b + NBUF, b)
        return carry

      lax.fori_loop(0, nblkr, _ring, 0)
      for b in range(NBUF):
        _wait(b)
      plsc.subcore_barrier()

      def _write(sizes):
        wdone = 0
        for n in sizes:
          pltpu.sync_copy(acc.at[pl.ds(woff + wdone, n)],
                          out_hbm.at[pl.ds(v_out + woff + wdone, n)])
          wdone += n

      _for_my_sizes(_write)

    for core in range(NC):
      @pl.when(cid == core)
      def _(core=core):
        half = len(stage_chunks) // NC
        for c in stage_chunks[core * half:(core + 1) * half]:
          do_chunk(c)

  agg_t = jax.ShapeDtypeStruct((STAGE_ROWS[stage], D), jnp.float32)
  if stage == 0:
    out_type = (agg_t, jax.ShapeDtypeStruct((128,), jnp.int32))
    extra_scratch = [pltpu.VMEM((SCAN_PER_TILE,), jnp.int32)]
  else:
    out_type = agg_t
    extra_scratch = []
  return pl.kernel(
      body,
      out_type=out_type,
      mesh=mesh,
      compiler_params=pltpu.CompilerParams(needs_layout_passes=False),
      scratch_types=[
          pltpu.VMEM((SB, D), jnp.float32),
          pltpu.VMEM((SB, D), jnp.float32),
          pltpu.VMEM((SB, D), jnp.float32),
          pltpu.VMEM((SB,), jnp.int32),
          pltpu.VMEM((SB,), jnp.int32),
          pltpu.VMEM((SB,), jnp.int32),
          pltpu.VMEM((1, 128), jnp.int32),
      ] + extra_scratch + [
          pltpu.VMEM((128,), jnp.int32),
          pltpu.VMEM((NS, 128), jnp.int32),
          pltpu.VMEM((32, D), jnp.float32),
          pltpu.SemaphoreType.DMA,
          pltpu.SemaphoreType.DMA,
          pltpu.SemaphoreType.DMA,
          pltpu.SemaphoreType.DMA,
          pltpu.SemaphoreType.DMA,
          pltpu.SemaphoreType.DMA,
          pltpu.VMEM_SHARED((NS, 128), jnp.int32),
          pltpu.VMEM_SHARED((ACC_ROWS, D), jnp.float32),
      ],
  )


BR = 3200


def _mlp_body(e_ref, a_ref, g_ref, w1_ref, b1_ref, w2_ref, b2_ref,
              gm_ref, bt_ref, o_ref):
  w1 = w1_ref[...]
  x = jnp.dot(e_ref[...], w1[0:D], preferred_element_type=jnp.float32)
  x = x + jnp.dot(a_ref[...], w1[D:2 * D],
                  preferred_element_type=jnp.float32)
  g = jnp.dot(g_ref[...], w1[2 * D:3 * D],
              preferred_element_type=jnp.float32)
  h = jnp.maximum(x + g + b1_ref[...], 0.0)
  h = jnp.maximum(
      jnp.dot(h, w2_ref[...], preferred_element_type=jnp.float32)
      + b2_ref[...], 0.0)
  m = jnp.mean(h, axis=-1, keepdims=True)
  cdev = h - m
  var = jnp.mean(cdev * cdev, axis=-1, keepdims=True)
  o_ref[...] = (cdev * lax.rsqrt(var + LN_EPS)) * gm_ref[...] + bt_ref[...]


def _mlp_body_alias(p_ref, e_ref, a_ref, g_ref, w1_ref, b1_ref, w2_ref,
                    b2_ref, gm_ref, bt_ref, o_ref):
  del p_ref
  _mlp_body(e_ref, a_ref, g_ref, w1_ref, b1_ref, w2_ref, b2_ref,
            gm_ref, bt_ref, o_ref)


def _tc_mlp_stage(stage, prev, edges, agg, globals_, W1, b1, W2, b2,
                  gamma, beta):
  nblocks = pl.cdiv(STAGE_ROWS[stage], BR)
  off = STAGE_BASE[stage] // BR
  full = lambda shape: pl.BlockSpec(shape, lambda i: (0, 0))
  row_spec = pl.BlockSpec((BR, D), lambda i: (i + off, 0))
  in_specs = [
      row_spec,
      pl.BlockSpec((BR, D), lambda i: (i, 0)),
      full((1, D)),
      full((3 * D, D)),
      full((1, D)),
      full((D, D)),
      full((1, D)),
      full((1, D)),
      full((1, D)),
  ]
  args = [edges, agg, globals_, W1, b1, W2, b2, gamma, beta]
  body = _mlp_body
  aliases = {}
  if stage > 0:
    in_specs = [pl.BlockSpec(memory_space=pl.ANY)] + in_specs
    args = [prev] + args
    body = _mlp_body_alias
    aliases = {0: 0}
  return pl.pallas_call(
      body,
      grid=(nblocks,),
      in_specs=in_specs,
      out_specs=row_spec,
      out_shape=jax.ShapeDtypeStruct((E, D), jnp.float32),
      input_output_aliases=aliases,
  )(*args)


def kernel(edges, nodes, globals_, segment_ids, num, W1, b1, W2, b2,
           gamma, beta):
  del num
  row = lambda v: v.reshape(1, D)
  wargs = (globals_, W1, row(b1), W2, row(b2), row(gamma), row(beta))
  agg0, rs = _make_sc_segment_sum(0)(nodes, segment_ids)
  aggs = [agg0] + [_make_sc_segment_sum(s)(nodes, segment_ids, rs)
                   for s in range(1, NSTAGES)]
  out = None
  for s in range(NSTAGES):
    out = _tc_mlp_stage(s, out, edges, aggs[s], *wargs)
  return out


# speedup vs baseline: 1.1003x; 1.0811x over previous
import functools

import jax
import jax.numpy as jnp
from jax import lax
from jax.experimental import pallas as pl
from jax.experimental.pallas import tpu as pltpu
from jax.experimental.pallas import tpu_sc as plsc

N = 100000
E = 50000
D = 128
LN_EPS = 1e-3

NC = 2
NS = 16
L = 16

NCHUNKS = 6
CB = 8448
CHUNK_LO = tuple(c * CB for c in range(NCHUNKS))
ACC_ROWS = 8576
DUMP = CB
CSW = CB // NS
LAST_REM = E - (NCHUNKS - 1) * CB
LAST_CSW = 488
LAST_TAIL = LAST_REM - (NS - 1) * LAST_CSW
STAGE_CHUNKS = ((0, 1, 2, 3, 4, 5),)
NSTAGES = len(STAGE_CHUNKS)
STAGE_BASE = tuple(CHUNK_LO[sc[0]] for sc in STAGE_CHUNKS)
STAGE_ROWS = tuple(
    min(E, CHUNK_LO[sc[-1]] + CB) - CHUNK_LO[sc[0]] for sc in STAGE_CHUNKS)

SCAN_MAIN = 99840
SCAN_PER_TILE = SCAN_MAIN // NS
SCAN_TAIL = N - SCAN_MAIN
SB = 128
NBUF = 3


@functools.lru_cache(maxsize=NSTAGES)
def _make_sc_segment_sum(stage):
  mesh = plsc.VectorSubcoreMesh(core_axis_name="c", subcore_axis_name="s",
                                num_cores=NC, num_subcores=NS)
  stage_chunks = STAGE_CHUNKS[stage]
  edges_needed = ([CHUNK_LO[c] for c in stage_chunks]
                  + [CHUNK_LO[stage_chunks[-1]] + CB])
  thresholds = (tuple(k * CB for k in range(1, NCHUNKS) if k * CB < E)
                if stage == 0 else ())

  def body(*refs):
    if stage == 0:
      (nodes_hbm, ids_hbm, out_hbm, rs_hbm,
       rows_v0, rows_v1, rows_v2, idsv0, idsv1, idsv2, idx_r,
       idscan_v, cnt_v, call_v, zeros_v,
       sem_r0, sem_r1, sem_r2, sem_i0, sem_i1, sem_i2,
       cnt_sh, acc) = refs
    else:
      (nodes_hbm, ids_hbm, rs_hbm, out_hbm,
       rows_v0, rows_v1, rows_v2, idsv0, idsv1, idsv2, idx_r,
       cnt_v, call_v, zeros_v,
       sem_r0, sem_r1, sem_r2, sem_i0, sem_i1, sem_i2,
       cnt_sh, acc) = refs
    rows_bufs = (rows_v0, rows_v1, rows_v2)
    ids_bufs = (idsv0, idsv1, idsv2)
    sems_r = (sem_r0, sem_r1, sem_r2)
    sems_i = (sem_i0, sem_i1, sem_i2)
    cid = lax.axis_index("c")
    sid = lax.axis_index("s")

    zvec = jnp.zeros((L,), jnp.float32)

    def _zrow(r, carry):
      for j in range(D // L):
        zeros_v[r, pl.ds(j * L, L)] = zvec
      return carry

    lax.fori_loop(0, zeros_v.shape[0], _zrow, 0)

    one = jnp.ones((L,), jnp.int32)
    zero = jnp.zeros((L,), jnp.int32)
    z = jnp.zeros((L,), jnp.int32)

    if stage == 0:
      nb = len(thresholds)
      base = pl.multiple_of(sid * SCAN_PER_TILE, 8)
      pltpu.sync_copy(ids_hbm.at[pl.ds(base, SCAN_PER_TILE)], idscan_v)

      def _count(i, accs):
        v = idscan_v[pl.ds(i * L, L)]
        return tuple(accs[k] + jnp.where(v < thresholds[k], one, zero)
                     for k in range(nb))

      cnts = lax.fori_loop(0, SCAN_PER_TILE // L, _count,
                           tuple(z for _ in range(nb)))
      for k in range(nb):
        cnt_v[pl.ds(k * L, L)] = cnts[k]
      pltpu.sync_copy(cnt_v, cnt_sh.at[sid])

      pltpu.sync_copy(ids_hbm.at[pl.ds(SCAN_MAIN, SCAN_TAIL)],
                      idscan_v.at[pl.ds(0, SCAN_TAIL)])

      def _count_tail(i, accs):
        v = idscan_v[pl.ds(i * L, L)]
        return tuple(accs[k] + jnp.where(v < thresholds[k], one, zero)
                     for k in range(nb))

      tails = lax.fori_loop(0, SCAN_TAIL // L, _count_tail,
                            tuple(z for _ in range(nb)))
      plsc.subcore_barrier()
      pltpu.sync_copy(cnt_sh, call_v)

      sums = list(tails)
      for s in range(NS):
        for k in range(nb):
          sums[k] = sums[k] + call_v[s, pl.ds(k * L, L)]
      totals = [jnp.sum(sums[k]) for k in range(nb)]
      for k in range(nb):
        cnt_v[pl.ds(k * L, L)] = totals[k] + z

      @pl.when((cid == 0) & (sid == 0))
      def _():
        pltpu.sync_copy(cnt_v, rs_hbm)

      def _boundary(k):
        return totals[k]
    else:
      pltpu.sync_copy(rs_hbm, cnt_v)

      def _boundary(k):
        return cnt_v[pl.ds(k * L, L)][0]

    bounds = []
    for v in edges_needed:
      if v <= 0:
        bounds.append(jnp.int32(0))
      elif v >= E:
        bounds.append(jnp.int32(N))
      else:
        bounds.append(_boundary(v // CB - 1))

    iota = lax.iota(jnp.int32, L)
    dump_vec = jnp.full((L,), DUMP, jnp.int32)

    def _wblocks(total):
      return (SB,) * (total // SB) + (
          (total % SB,) if total % SB else ())

    def _strip_sizes(c):
      if CHUNK_LO[c] + CB <= E:
        return CSW, _wblocks(CSW), _wblocks(CSW)
      return LAST_CSW, _wblocks(LAST_CSW), _wblocks(LAST_TAIL)

    def do_chunk(c):
      cc = stage_chunks.index(c)
      v_lo = CHUNK_LO[c]
      v_out = v_lo - STAGE_BASE[stage]
      cs = CB
      lo, hi = bounds[cc], bounds[cc + 1]
      csw, sizes_main, sizes_last = _strip_sizes(c)
      woff = pl.multiple_of(sid * csw, 8)

      def _for_my_sizes(fn):
        @pl.when(sid < NS - 1)
        def _():
          fn(sizes_main)

        @pl.when(sid == NS - 1)
        def _():
          fn(sizes_last)

      def _zero(sizes):
        done = 0
        for n in sizes:
          zdone = 0
          while zdone < n:
            zn = min(n - zdone, zeros_v.shape[0])
            pltpu.sync_copy(zeros_v.at[pl.ds(0, zn)],
                            acc.at[pl.ds(woff + done + zdone, zn)])
            zdone += zn
          done += n

      _for_my_sizes(_zero)
      plsc.subcore_barrier()

      lo8 = lo - lax.rem(lo, 8)
      span = hi - lo8
      sub = ((span + 127) // 128) * 8
      a_t = lo8 + sid * sub
      b_t = a_t + sub
      nblkr = (sub + NBUF * SB - 1) // (NBUF * SB)

      def _start_for(j):
        return pl.multiple_of(jnp.minimum(a_t + j * SB, N - SB), 8)

      def _issue(j, b):
        st = _start_for(j)
        pltpu.async_copy(ids_hbm.at[pl.ds(st, SB)], ids_bufs[b], sems_i[b])
        pltpu.async_copy(nodes_hbm.at[pl.ds(st, SB)], rows_bufs[b],
                         sems_r[b])

      def _wait(b):
        pltpu.make_async_copy(ids_hbm.at[pl.ds(0, SB)], ids_bufs[b],
                              sems_i[b]).wait()
        pltpu.make_async_copy(nodes_hbm.at[pl.ds(0, SB)], rows_bufs[b],
                              sems_r[b]).wait()

      def _process(j, b):
        nominal = a_t + j * SB
        start = _start_for(j)
        for i in range(SB // L):
          v = ids_bufs[b][pl.ds(i * L, L)]
          local = v - v_lo
          rowid = iota + (start + i * L)
          m = ((local >= 0) & (local < cs)
               & (rowid >= nominal) & (rowid < b_t))
          idx = jnp.where(m, local, dump_vec)
          idx_r[0, pl.ds(i * L, L)] = idx
        pltpu.sync_copy(rows_bufs[b], acc.at[idx_r.at[0]], add=True)

      for b in range(NBUF):
        _issue(b, b)

      def _ring(jr, carry):
        j = NBUF * jr
        for b in range(NBUF):
          _wait(b)
          _process(j + b, b)
          _issue(j + b + NBUF, b)
        return carry

      lax.fori_loop(0, nblkr, _ring, 0)
      for b in range(NBUF):
        _wait(b)
      plsc.subcore_barrier()

      def _write(sizes):
        wdone = 0
        for n in sizes:
          pltpu.sync_copy(acc.at[pl.ds(woff + wdone, n)],
                          out_hbm.at[pl.ds(v_out + woff + wdone, n)])
          wdone += n

      _for_my_sizes(_write)

    for core in range(NC):
      @pl.when(cid == core)
      def _(core=core):
        half = len(stage_chunks) // NC
        for c in stage_chunks[core * half:(core + 1) * half]:
          do_chunk(c)

  agg_t = jax.ShapeDtypeStruct((STAGE_ROWS[stage], D), jnp.float32)
  if stage == 0:
    out_type = (agg_t, jax.ShapeDtypeStruct((128,), jnp.int32))
    extra_scratch = [pltpu.VMEM((SCAN_PER_TILE,), jnp.int32)]
  else:
    out_type = agg_t
    extra_scratch = []
  return pl.kernel(
      body,
      out_type=out_type,
      mesh=mesh,
      compiler_params=pltpu.CompilerParams(needs_layout_passes=False),
      scratch_types=[
          pltpu.VMEM((SB, D), jnp.float32),
          pltpu.VMEM((SB, D), jnp.float32),
          pltpu.VMEM((SB, D), jnp.float32),
          pltpu.VMEM((SB,), jnp.int32),
          pltpu.VMEM((SB,), jnp.int32),
          pltpu.VMEM((SB,), jnp.int32),
          pltpu.VMEM((1, 128), jnp.int32),
      ] + extra_scratch + [
          pltpu.VMEM((128,), jnp.int32),
          pltpu.VMEM((NS, 128), jnp.int32),
          pltpu.VMEM((32, D), jnp.float32),
          pltpu.SemaphoreType.DMA,
          pltpu.SemaphoreType.DMA,
          pltpu.SemaphoreType.DMA,
          pltpu.SemaphoreType.DMA,
          pltpu.SemaphoreType.DMA,
          pltpu.SemaphoreType.DMA,
          pltpu.VMEM_SHARED((NS, 128), jnp.int32),
          pltpu.VMEM_SHARED((ACC_ROWS, D), jnp.float32),
      ],
  )


BR = 5000


def _mlp_body(e_ref, a_ref, g_ref, w1_ref, b1_ref, w2_ref, b2_ref,
              gm_ref, bt_ref, o_ref):
  w1 = w1_ref[...]
  x = jnp.dot(e_ref[...], w1[0:D], preferred_element_type=jnp.float32)
  x = x + jnp.dot(a_ref[...], w1[D:2 * D],
                  preferred_element_type=jnp.float32)
  g = jnp.dot(g_ref[...], w1[2 * D:3 * D],
              preferred_element_type=jnp.float32)
  h = jnp.maximum(x + g + b1_ref[...], 0.0)
  h = jnp.maximum(
      jnp.dot(h, w2_ref[...], preferred_element_type=jnp.float32)
      + b2_ref[...], 0.0)
  m = jnp.mean(h, axis=-1, keepdims=True)
  cdev = h - m
  var = jnp.mean(cdev * cdev, axis=-1, keepdims=True)
  o_ref[...] = (cdev * lax.rsqrt(var + LN_EPS)) * gm_ref[...] + bt_ref[...]


def _mlp_body_alias(p_ref, e_ref, a_ref, g_ref, w1_ref, b1_ref, w2_ref,
                    b2_ref, gm_ref, bt_ref, o_ref):
  del p_ref
  _mlp_body(e_ref, a_ref, g_ref, w1_ref, b1_ref, w2_ref, b2_ref,
            gm_ref, bt_ref, o_ref)


def _tc_mlp_stage(stage, prev, edges, agg, globals_, W1, b1, W2, b2,
                  gamma, beta):
  nblocks = pl.cdiv(STAGE_ROWS[stage], BR)
  off = STAGE_BASE[stage] // BR
  full = lambda shape: pl.BlockSpec(shape, lambda i: (0, 0))
  row_spec = pl.BlockSpec((BR, D), lambda i: (i + off, 0))
  in_specs = [
      row_spec,
      pl.BlockSpec((BR, D), lambda i: (i, 0)),
      full((1, D)),
      full((3 * D, D)),
      full((1, D)),
      full((D, D)),
      full((1, D)),
      full((1, D)),
      full((1, D)),
  ]
  args = [edges, agg, globals_, W1, b1, W2, b2, gamma, beta]
  body = _mlp_body
  aliases = {}
  if stage > 0:
    in_specs = [pl.BlockSpec(memory_space=pl.ANY)] + in_specs
    args = [prev] + args
    body = _mlp_body_alias
    aliases = {0: 0}
  return pl.pallas_call(
      body,
      grid=(nblocks,),
      in_specs=in_specs,
      out_specs=row_spec,
      out_shape=jax.ShapeDtypeStruct((E, D), jnp.float32),
      input_output_aliases=aliases,
  )(*args)


def kernel(edges, nodes, globals_, segment_ids, num, W1, b1, W2, b2,
           gamma, beta):
  del num
  row = lambda v: v.reshape(1, D)
  wargs = (globals_, W1, row(b1), W2, row(b2), row(gamma), row(beta))
  agg0, rs = _make_sc_segment_sum(0)(nodes, segment_ids)
  aggs = [agg0] + [_make_sc_segment_sum(s)(nodes, segment_ids, rs)
                   for s in range(1, NSTAGES)]
  out = None
  for s in range(NSTAGES):
    out = _tc_mlp_stage(s, out, edges, aggs[s], *wargs)
  return out
